# baseline (speedup 1.0000x reference)
import jax
import jax.numpy as jnp
from jax.experimental import pallas as pl

_N = 524288
_BLK = 2048


def _put_kernel(x_ref, o_ref):
    i = pl.program_id(0)
    o_ref[...] = x_ref[...]

    @pl.when(i == 0)
    def _():
        ti = (jax.lax.broadcasted_iota(jnp.int32, (6, 8), 0) * 8
              + jax.lax.broadcasted_iota(jnp.int32, (6, 8), 1))
        t = ti.astype(jnp.float32)
        o_ref[0:7, :, :] = jnp.broadcast_to(t[None], (7, 6, 8))


def kernel(x):
    return pl.pallas_call(
        _put_kernel,
        grid=(_N // _BLK,),
        in_specs=[pl.BlockSpec((_BLK, 6, 8), lambda i: (i, 0, 0))],
        out_specs=pl.BlockSpec((_BLK, 6, 8), lambda i: (i, 0, 0)),
        out_shape=jax.ShapeDtypeStruct((_N, 6, 8), jnp.float32),
    )(x)


# speedup vs baseline: 1.0003x; 1.0003x over previous
import jax
import jax.numpy as jnp
from jax.experimental import pallas as pl

_N = 524288
_ROWS = _N * 48 // 128
_BLK = 4096


def _put_kernel(x_ref, o_ref):
    i = pl.program_id(0)
    o_ref[...] = x_ref[...]

    @pl.when(i == 0)
    def _():
        r = jax.lax.broadcasted_iota(jnp.int32, (8, 128), 0)
        c = jax.lax.broadcasted_iota(jnp.int32, (8, 128), 1)
        l = r * 128 + c
        val = (l - 48 * (l // 48)).astype(jnp.float32)
        o_ref[0:8, :] = jnp.where(l < 336, val, x_ref[0:8, :])


def kernel(x):
    xr = jnp.reshape(x, (_ROWS, 128))
    y = pl.pallas_call(
        _put_kernel,
        grid=(_ROWS // _BLK,),
        in_specs=[pl.BlockSpec((_BLK, 128), lambda i: (i, 0))],
        out_specs=pl.BlockSpec((_BLK, 128), lambda i: (i, 0)),
        out_shape=jax.ShapeDtypeStruct((_ROWS, 128), jnp.float32),
    )(xr)
    return jnp.reshape(y, (_N, 6, 8))


# speedup vs baseline: 43.0223x; 43.0082x over previous
import jax
import jax.numpy as jnp
from jax.experimental import pallas as pl

_N = 524288
_BLKC = 8192


def _put_kernel(x_ref, o_ref):
    i = pl.program_id(0)

    @pl.when(i > 0)
    def _():
        o_ref[...] = x_ref[...]

    @pl.when(i == 0)
    def _():
        lane = jax.lax.broadcasted_iota(jnp.int32, (48, _BLKC), 1)
        row = jax.lax.broadcasted_iota(jnp.int32, (48, _BLKC), 0)
        o_ref[...] = jnp.where(lane < 7, row.astype(jnp.float32), x_ref[...])


def kernel(x):
    xt = jnp.transpose(x, (1, 2, 0)).reshape(48, _N)
    y = pl.pallas_call(
        _put_kernel,
        grid=(_N // _BLKC,),
        in_specs=[pl.BlockSpec((48, _BLKC), lambda i: (0, i))],
        out_specs=pl.BlockSpec((48, _BLKC), lambda i: (0, i)),
        out_shape=jax.ShapeDtypeStruct((48, _N), jnp.float32),
    )(xt)
    return jnp.transpose(y.reshape(6, 8, _N), (2, 0, 1))


# speedup vs baseline: 50.9388x; 1.1840x over previous
import jax
import jax.numpy as jnp
from jax.experimental import pallas as pl

_N = 524288
_BLKC = 16384


def _put_kernel(x_ref, o_ref):
    i = pl.program_id(0)

    @pl.when(i > 0)
    def _():
        o_ref[...] = x_ref[...]

    @pl.when(i == 0)
    def _():
        lane = jax.lax.broadcasted_iota(jnp.int32, (48, _BLKC), 1)
        row = jax.lax.broadcasted_iota(jnp.int32, (48, _BLKC), 0)
        o_ref[...] = jnp.where(lane < 7, row.astype(jnp.float32), x_ref[...])


def kernel(x):
    xt = jnp.transpose(x, (1, 2, 0)).reshape(48, _N)
    y = pl.pallas_call(
        _put_kernel,
        grid=(_N // _BLKC,),
        in_specs=[pl.BlockSpec((48, _BLKC), lambda i: (0, i))],
        out_specs=pl.BlockSpec((48, _BLKC), lambda i: (0, i)),
        out_shape=jax.ShapeDtypeStruct((48, _N), jnp.float32),
    )(xt)
    return jnp.transpose(y.reshape(6, 8, _N), (2, 0, 1))


# speedup vs baseline: 52.8328x; 1.0372x over previous
import jax
import jax.numpy as jnp
from jax.experimental import pallas as pl

_N = 524288
_BLKC = 32768


def _put_kernel(x_ref, o_ref):
    i = pl.program_id(0)

    @pl.when(i > 0)
    def _():
        o_ref[...] = x_ref[...]

    @pl.when(i == 0)
    def _():
        lane = jax.lax.broadcasted_iota(jnp.int32, (48, _BLKC), 1)
        row = jax.lax.broadcasted_iota(jnp.int32, (48, _BLKC), 0)
        o_ref[...] = jnp.where(lane < 7, row.astype(jnp.float32), x_ref[...])


def kernel(x):
    xt = jnp.transpose(x, (1, 2, 0)).reshape(48, _N)
    y = pl.pallas_call(
        _put_kernel,
        grid=(_N // _BLKC,),
        in_specs=[pl.BlockSpec((48, _BLKC), lambda i: (0, i))],
        out_specs=pl.BlockSpec((48, _BLKC), lambda i: (0, i)),
        out_shape=jax.ShapeDtypeStruct((48, _N), jnp.float32),
    )(xt)
    return jnp.transpose(y.reshape(6, 8, _N), (2, 0, 1))


# speedup vs baseline: 53.5636x; 1.0138x over previous
import jax
import jax.numpy as jnp
from jax.experimental import pallas as pl

_N = 524288
_BLKC = 65536


def _put_kernel(x_ref, o_ref):
    i = pl.program_id(0)

    @pl.when(i > 0)
    def _():
        o_ref[...] = x_ref[...]

    @pl.when(i == 0)
    def _():
        lane = jax.lax.broadcasted_iota(jnp.int32, (48, _BLKC), 1)
        row = jax.lax.broadcasted_iota(jnp.int32, (48, _BLKC), 0)
        o_ref[...] = jnp.where(lane < 7, row.astype(jnp.float32), x_ref[...])


def kernel(x):
    xt = jnp.transpose(x, (1, 2, 0)).reshape(48, _N)
    y = pl.pallas_call(
        _put_kernel,
        grid=(_N // _BLKC,),
        in_specs=[pl.BlockSpec((48, _BLKC), lambda i: (0, i))],
        out_specs=pl.BlockSpec((48, _BLKC), lambda i: (0, i)),
        out_shape=jax.ShapeDtypeStruct((48, _N), jnp.float32),
    )(xt)
    return jnp.transpose(y.reshape(6, 8, _N), (2, 0, 1))
